# baseline (speedup 1.0000x reference)
import numpy as np
import jax
import jax.numpy as jnp
from jax.experimental import pallas as pl


def _pe_div_term(dim, lambda_max=10000.0, lambda_min=1e-05):
    base = lambda_max / (2 * np.pi)
    scale = lambda_min / lambda_max
    return (base * scale ** (np.arange(0, dim, 2) / dim)).astype(np.float32)


def _body(idx_ref, x_ref, tab_ref, out_ref):
    bt = idx_ref.shape[0]
    vocab = tab_ref.shape[0]
    dh = x_ref.shape[1]
    idx = idx_ref[:, :]
    iota = jax.lax.broadcasted_iota(jnp.int32, (bt, vocab), 1)
    onehot = jnp.where(iota == idx, 1.0, 0.0)
    g = jnp.dot(onehot, tab_ref[:, :], preferred_element_type=jnp.float32,
                precision=jax.lax.Precision.HIGHEST)
    x = x_ref[:, :]
    out_ref[:, :dh] = g[:, :dh] + jnp.sin(x)
    out_ref[:, dh:] = g[:, dh:] + jnp.cos(x)


def kernel(tgt, pos_index, tgt_token_embedding):
    b, t = tgt.shape
    vocab, dim = tgt_token_embedding.shape
    n = b * t
    dh = dim // 2
    bt = 512

    idx = tgt.reshape(n, 1).astype(jnp.int32)
    pos = pos_index.reshape(n, 1).astype(jnp.float32)
    div = jnp.asarray(_pe_div_term(dim)).reshape(1, dh)
    x = pos / div

    out = pl.pallas_call(
        _body,
        grid=(n // bt,),
        in_specs=[
            pl.BlockSpec((bt, 1), lambda i: (i, 0)),
            pl.BlockSpec((bt, dh), lambda i: (i, 0)),
            pl.BlockSpec((vocab, dim), lambda i: (0, 0)),
        ],
        out_specs=pl.BlockSpec((bt, dim), lambda i: (i, 0)),
        out_shape=jax.ShapeDtypeStruct((n, dim), jnp.float32),
    )(idx, x, tgt_token_embedding)
    return out.reshape(b, t, dim)


# speedup vs baseline: 1.3551x; 1.3551x over previous
import numpy as np
import jax
import jax.numpy as jnp
from jax.experimental import pallas as pl


def _pe_div_term(dim, lambda_max=10000.0, lambda_min=1e-05):
    base = lambda_max / (2 * np.pi)
    scale = lambda_min / lambda_max
    return (base * scale ** (np.arange(0, dim, 2) / dim)).astype(np.float32)


def _body(idx_ref, x_ref, thi_ref, tlo_ref, out_ref):
    bt = idx_ref.shape[0]
    vocab = thi_ref.shape[0]
    dh = x_ref.shape[1]
    idx = idx_ref[:, :]
    iota = jax.lax.broadcasted_iota(jnp.int32, (bt, vocab), 1)
    onehot = jnp.where(iota == idx, 1.0, 0.0).astype(jnp.bfloat16)
    g = jnp.dot(onehot, thi_ref[:, :], preferred_element_type=jnp.float32)
    g = g + jnp.dot(onehot, tlo_ref[:, :], preferred_element_type=jnp.float32)
    x = x_ref[:, :]
    out_ref[:, :dh] = g[:, :dh] + jnp.sin(x)
    out_ref[:, dh:] = g[:, dh:] + jnp.cos(x)


def kernel(tgt, pos_index, tgt_token_embedding):
    b, t = tgt.shape
    vocab, dim = tgt_token_embedding.shape
    n = b * t
    dh = dim // 2
    bt = 512

    idx = tgt.reshape(n, 1).astype(jnp.int32)
    pos = pos_index.reshape(n, 1).astype(jnp.float32)
    thi = tgt_token_embedding.astype(jnp.bfloat16)
    tlo = (tgt_token_embedding - thi.astype(jnp.float32)).astype(jnp.bfloat16)
    div = jnp.asarray(_pe_div_term(dim)).reshape(1, dh)
    x = pos / div

    out = pl.pallas_call(
        _body,
        grid=(n // bt,),
        in_specs=[
            pl.BlockSpec((bt, 1), lambda i: (i, 0)),
            pl.BlockSpec((bt, dh), lambda i: (i, 0)),
            pl.BlockSpec((vocab, dim), lambda i: (0, 0)),
            pl.BlockSpec((vocab, dim), lambda i: (0, 0)),
        ],
        out_specs=pl.BlockSpec((bt, dim), lambda i: (i, 0)),
        out_shape=jax.ShapeDtypeStruct((n, dim), jnp.float32),
    )(idx, x, thi, tlo)
    return out.reshape(b, t, dim)


# speedup vs baseline: 1.3885x; 1.0246x over previous
import numpy as np
import jax
import jax.numpy as jnp
from jax.experimental import pallas as pl


def _pe_div_term(dim, lambda_max=10000.0, lambda_min=1e-05):
    base = lambda_max / (2 * np.pi)
    scale = lambda_min / lambda_max
    return (base * scale ** (np.arange(0, dim, 2) / dim)).astype(np.float32)


def _body(idx_ref, x_ref, thi_ref, tlo_ref, out_ref):
    vocab = thi_ref.shape[0]
    dh = x_ref.shape[1]
    idx = idx_ref[0]
    bt = idx.shape[1]
    iota = jax.lax.broadcasted_iota(jnp.int32, (vocab, bt), 0)
    onehot_t = jnp.where(iota == idx, 1.0, 0.0).astype(jnp.bfloat16)
    dn = (((0,), (0,)), ((), ()))
    g = jax.lax.dot_general(onehot_t, thi_ref[:, :], dn,
                            preferred_element_type=jnp.float32)
    g = g + jax.lax.dot_general(onehot_t, tlo_ref[:, :], dn,
                                preferred_element_type=jnp.float32)
    x = x_ref[:, :]
    out_ref[:, :dh] = g[:, :dh] + jnp.sin(x)
    out_ref[:, dh:] = g[:, dh:] + jnp.cos(x)


def kernel(tgt, pos_index, tgt_token_embedding):
    b, t = tgt.shape
    vocab, dim = tgt_token_embedding.shape
    n = b * t
    dh = dim // 2
    bt = 512

    idx = tgt.reshape(n // bt, 1, bt).astype(jnp.int32)
    pos = pos_index.reshape(n, 1).astype(jnp.float32)
    thi = tgt_token_embedding.astype(jnp.bfloat16)
    tlo = (tgt_token_embedding - thi.astype(jnp.float32)).astype(jnp.bfloat16)
    div = jnp.asarray(_pe_div_term(dim)).reshape(1, dh)
    x = pos / div

    out = pl.pallas_call(
        _body,
        grid=(n // bt,),
        in_specs=[
            pl.BlockSpec((1, 1, bt), lambda i: (i, 0, 0)),
            pl.BlockSpec((bt, dh), lambda i: (i, 0)),
            pl.BlockSpec((vocab, dim), lambda i: (0, 0)),
            pl.BlockSpec((vocab, dim), lambda i: (0, 0)),
        ],
        out_specs=pl.BlockSpec((bt, dim), lambda i: (i, 0)),
        out_shape=jax.ShapeDtypeStruct((n, dim), jnp.float32),
    )(idx, x, thi, tlo)
    return out.reshape(b, t, dim)
